# baseline (speedup 1.0000x reference)
import functools

import jax
import jax.numpy as jnp
from jax import lax
from jax.experimental import pallas as pl
from jax.experimental.pallas import tpu as pltpu
from jax.experimental.pallas import tpu_sc as plsc

N = 10000
K = 20
NSEG = 32
R = 256
RK = 256
C = 256
NP = 10240
NB = NP // R
NBK = NP // RK
DP = 128
NEG = float("-inf")
FBIG = 3.0e9

SC_NC = 2
SC_NS = 16
SC_L = 16
SC_NW = SC_NC * SC_NS
SC_T = 16
SC_GRP = 80


def _extract_topk(vals, gids):
    nv, ni = [], []
    for _ in range(K):
        m = jnp.max(vals, axis=1, keepdims=True)
        eq = vals == m
        g = jnp.min(jnp.where(eq, gids, FBIG), axis=1, keepdims=True)
        nv.append(m)
        ni.append(g)
        vals = jnp.where(eq & (gids == g), NEG, vals)
    return jnp.concatenate(nv, axis=1), jnp.concatenate(ni, axis=1)


def _knn_lin_body(cs_ref, ce_ref, x_ref, xT_ref, brow_ref, bcolT_ref,
                  wb_ref, wab_ref, bias_ref, idx_ref, y_ref, z_ref):
    b = pl.program_id(0)
    r0 = pl.multiple_of(b * RK, RK)
    xr = x_ref[pl.ds(r0, RK), :]
    sq_r = jnp.sum(xr * xr, axis=1, keepdims=True)
    br = brow_ref[...]

    y_ref[...] = jnp.dot(xr, wb_ref[...], preferred_element_type=jnp.float32)
    z_ref[...] = (jnp.dot(xr, wab_ref[...], preferred_element_type=jnp.float32)
                  + bias_ref[...])

    topv0 = jnp.full((RK, K), NEG, dtype=jnp.float32)
    topi0 = lax.broadcasted_iota(jnp.int32, (RK, K), 1).astype(jnp.float32)

    def col_step(cb, carry):
        topv, topi = carry
        c0 = pl.multiple_of(cb * C, C)
        xcT = xT_ref[:, pl.ds(c0, C)]
        sq_c = jnp.sum(xcT * xcT, axis=0, keepdims=True)
        t0 = jnp.dot(xr, xcT, preferred_element_type=jnp.float32)
        dist = (sq_r - 2.0 * t0) + sq_c
        bc = bcolT_ref[:, pl.ds(c0, C)]
        s = jnp.where(br == bc, -dist, NEG)
        gid = (c0 + lax.broadcasted_iota(jnp.int32, (RK, C), 1)
               ).astype(jnp.float32)

        sv, si = _extract_topk(s, gid)
        vals = jnp.concatenate([topv, sv], axis=1)
        gids = jnp.concatenate([topi, si], axis=1)
        return _extract_topk(vals, gids)

    topv, topi = lax.fori_loop(cs_ref[b], ce_ref[b], col_step, (topv0, topi0))
    idx_ref[...] = topi.astype(jnp.int32)


def _knn_and_linear(x_pad, xT, brow, bcolT, cs, ce, Wb, Wab, bias, fo):
    return pl.pallas_call(
        _knn_lin_body,
        grid=(NBK,),
        in_specs=[
            pl.BlockSpec(memory_space=pltpu.SMEM),
            pl.BlockSpec(memory_space=pltpu.SMEM),
            pl.BlockSpec((NP, DP), lambda b: (0, 0)),
            pl.BlockSpec((DP, NP), lambda b: (0, 0)),
            pl.BlockSpec((RK, 1), lambda b: (b, 0)),
            pl.BlockSpec((1, NP), lambda b: (0, 0)),
            pl.BlockSpec((DP, fo), lambda b: (0, 0)),
            pl.BlockSpec((DP, fo), lambda b: (0, 0)),
            pl.BlockSpec((1, fo), lambda b: (0, 0)),
        ],
        out_specs=[
            pl.BlockSpec((RK, K), lambda b: (b, 0)),
            pl.BlockSpec((RK, fo), lambda b: (b, 0)),
            pl.BlockSpec((RK, fo), lambda b: (b, 0)),
        ],
        out_shape=[
            jax.ShapeDtypeStruct((NP, K), jnp.int32),
            jax.ShapeDtypeStruct((NP, fo), jnp.float32),
            jax.ShapeDtypeStruct((NP, fo), jnp.float32),
        ],
    )(cs, ce, x_pad, xT, brow, bcolT, Wb, Wab, bias)


def _gather_max_sc(y, z, idx_flat, fo):
    n_per_w = NP // SC_NW
    steps = n_per_w // SC_T
    ngrp = (SC_T * K) // SC_GRP
    fchunks = fo // SC_L
    mesh = plsc.VectorSubcoreMesh(core_axis_name="c", subcore_axis_name="s")

    @functools.partial(
        pl.kernel, mesh=mesh,
        out_type=jax.ShapeDtypeStruct((NP, fo), jnp.float32),
        scratch_types=[
            pltpu.VMEM((SC_T * K,), jnp.int32),
            pltpu.VMEM((SC_T * K, fo), jnp.float32),
            pltpu.VMEM((SC_T, fo), jnp.float32),
            pltpu.SemaphoreType.DMA,
        ],
    )
    def body(y_hbm, z_hbm, idx_hbm, out_hbm, idx_v, rows_v, acc_v, sem):
        wid = lax.axis_index("s") * SC_NC + lax.axis_index("c")
        base = wid * n_per_w

        def step(t, carry):
            n0 = base + t * SC_T
            pltpu.sync_copy(idx_hbm.at[pl.ds(n0 * K, SC_T * K)], idx_v)
            cps = [
                pltpu.async_copy(
                    y_hbm.at[idx_v.at[pl.ds(g * SC_GRP, SC_GRP)]],
                    rows_v.at[pl.ds(g * SC_GRP, SC_GRP)],
                    sem,
                )
                for g in range(ngrp)
            ]
            for cp in cps:
                cp.wait()
            pltpu.sync_copy(z_hbm.at[pl.ds(n0, SC_T)], acc_v)

            def inner(q, c2):
                n = q // fchunks
                f = (q % fchunks) * SC_L
                rbase = n * K
                a = rows_v[rbase, pl.ds(f, SC_L)]
                for kk in range(1, K):
                    a = jnp.maximum(a, rows_v[rbase + kk, pl.ds(f, SC_L)])
                acc_v[n, pl.ds(f, SC_L)] = acc_v[n, pl.ds(f, SC_L)] + a
                return c2

            lax.fori_loop(0, SC_T * fchunks, inner, 0)
            pltpu.sync_copy(acc_v, out_hbm.at[pl.ds(n0, SC_T)])
            return carry

        lax.fori_loop(0, steps, step, 0)

    return body(y, z, idx_flat)


def _linear_pool_body(slo_ref, shi_ref, cat_ref, lw_ref, lb_ref, brow_ref,
                      out_ref):
    b = pl.program_id(0)
    h = (jnp.dot(cat_ref[...], lw_ref[...], preferred_element_type=jnp.float32)
         + lb_ref[...])
    br = brow_ref[...]

    @pl.when(b == 0)
    def _():
        out_ref[...] = jnp.full(out_ref.shape, NEG, dtype=jnp.float32)

    def seg_step(s, carry):
        vals = jnp.where(br == s, h, NEG)
        m = jnp.max(vals, axis=0, keepdims=True)
        cur = out_ref[pl.ds(s, 1), :]
        out_ref[pl.ds(s, 1), :] = jnp.maximum(cur, m)
        return carry

    lax.fori_loop(slo_ref[b], shi_ref[b] + 1, seg_step, 0)


def _linear_pool(cat, lW, lb2, brow, slo, shi):
    return pl.pallas_call(
        _linear_pool_body,
        grid=(NB,),
        in_specs=[
            pl.BlockSpec(memory_space=pltpu.SMEM),
            pl.BlockSpec(memory_space=pltpu.SMEM),
            pl.BlockSpec((R, 512), lambda b: (b, 0)),
            pl.BlockSpec((512, 1024), lambda b: (0, 0)),
            pl.BlockSpec((1, 1024), lambda b: (0, 0)),
            pl.BlockSpec((R, 1), lambda b: (b, 0)),
        ],
        out_specs=pl.BlockSpec((NSEG, 1024), lambda b: (0, 0)),
        out_shape=jax.ShapeDtypeStruct((NSEG, 1024), jnp.float32),
        compiler_params=pltpu.CompilerParams(
            dimension_semantics=("arbitrary",)),
    )(slo, shi, cat, lW, lb2, brow)


def _head_body(p_ref, w1_ref, b1_ref, w2_ref, b2_ref, w3_ref, b3_ref, out_ref):
    p = p_ref[...]
    h = jnp.maximum(jnp.dot(p, w1_ref[...], preferred_element_type=jnp.float32)
                    + b1_ref[...], 0.0)
    h = jnp.maximum(jnp.dot(h, w2_ref[...], preferred_element_type=jnp.float32)
                    + b2_ref[...], 0.0)
    h = (jnp.dot(h, w3_ref[...], preferred_element_type=jnp.float32)
         + b3_ref[...])
    m = jnp.max(h, axis=1, keepdims=True)
    sh = h - m
    lse = jnp.log(jnp.sum(jnp.exp(sh), axis=1, keepdims=True))
    out_ref[...] = sh - lse


def _head(pooled, m1W, m1b, m2W, m2b, m3W, m3b):
    return pl.pallas_call(
        _head_body,
        out_shape=jax.ShapeDtypeStruct((NSEG, 40), jnp.float32),
    )(pooled, m1W, m1b.reshape(1, -1), m2W, m2b.reshape(1, -1),
      m3W, m3b.reshape(1, -1))


def _edge_layer(x_pad, brow, bcolT, cs, ce, W, bvec, fi, fo):
    fp = ((fo + 127) // 128) * 128
    Wa = W[:fi]
    Wb = W[fi:]
    Wb_p = jnp.zeros((DP, fp), jnp.float32).at[:fi, :fo].set(Wb)
    Wab_p = jnp.zeros((DP, fp), jnp.float32).at[:fi, :fo].set(Wa - Wb)
    bvec_p = jnp.zeros((1, fp), jnp.float32).at[0, :fo].set(bvec)
    xT = x_pad.T
    idx, y, z = _knn_and_linear(x_pad, xT, brow, bcolT, cs, ce,
                                Wb_p, Wab_p, bvec_p, fp)
    out = _gather_max_sc(y, z, idx.reshape(NP * K), fp)
    return out


def kernel(pos, batch, W1, b1, W2, b2, W3, b3, W4, b4, lW, lb,
           m1W, m1b, m2W, m2b, m3W, m3b):
    batch = batch.astype(jnp.int32)
    batch_pad = jnp.full((NP,), -1, jnp.int32).at[:N].set(batch)
    brow = batch_pad.reshape(NP, 1)
    bcolT = batch_pad.reshape(1, NP)

    seg_ids = jnp.arange(NSEG, dtype=jnp.int32)
    seg_start = jnp.searchsorted(batch, seg_ids, side="left").astype(jnp.int32)
    seg_end = jnp.searchsorted(batch, seg_ids, side="right").astype(jnp.int32)
    rbk = jnp.arange(NBK, dtype=jnp.int32) * RK
    firstk = batch[jnp.minimum(rbk, N - 1)]
    lastk = batch[jnp.minimum(rbk + RK - 1, N - 1)]
    cs = seg_start[firstk] // C
    ce = (seg_end[lastk] + C - 1) // C

    rb = jnp.arange(NB, dtype=jnp.int32) * R
    first = batch[jnp.minimum(rb, N - 1)]
    last = batch[jnp.minimum(rb + R - 1, N - 1)]

    x_pad = jnp.zeros((NP, DP), jnp.float32).at[:N, :3].set(pos)
    x1 = _edge_layer(x_pad, brow, bcolT, cs, ce, W1, b1, 3, 64)
    x2 = _edge_layer(x1, brow, bcolT, cs, ce, W2, b2, 64, 64)
    x3 = _edge_layer(x2, brow, bcolT, cs, ce, W3, b3, 64, 128)
    x4 = _edge_layer(x3, brow, bcolT, cs, ce, W4, b4, 128, 256)

    cat = jnp.concatenate([x1[:, :64], x2[:, :64], x3, x4], axis=1)
    pooled = _linear_pool(cat, lW, lb.reshape(1, -1), brow, first, last)
    return _head(pooled, m1W, m1b, m2W, m2b, m3W, m3b)


# speedup vs baseline: 2.4464x; 2.4464x over previous
import functools

import jax
import jax.numpy as jnp
from jax import lax
from jax.experimental import pallas as pl
from jax.experimental.pallas import tpu as pltpu
from jax.experimental.pallas import tpu_sc as plsc

N = 10000
K = 20
NSEG = 32
R = 256
RK = 256
C = 256
NP = 10240
NB = NP // R
NBK = NP // RK
DP = 128
NEG = float("-inf")
FBIG = 3.0e9

SC_NC = 2
SC_NS = 16
SC_L = 16
SC_NW = SC_NC * SC_NS
SC_T = 16
SC_GRP = 80


KP = 24


def _extract_topk_rows(vals, gids):
    nv, ni = [], []
    for _ in range(K):
        m = jnp.max(vals, axis=0, keepdims=True)
        eq = vals == m
        g = jnp.min(jnp.where(eq, gids, FBIG), axis=0, keepdims=True)
        nv.append(m)
        ni.append(g)
        vals = jnp.where(eq & (gids == g), NEG, vals)
    r = nv[0].shape[1]
    nv.append(jnp.full((KP - K, r), NEG, dtype=jnp.float32))
    ni.append(jnp.full((KP - K, r), FBIG, dtype=jnp.float32))
    return jnp.concatenate(nv, axis=0), jnp.concatenate(ni, axis=0)


def _knn_lin_body(cs_ref, ce_ref, x_ref, xT_ref, batchF_ref, bcolT_ref,
                  wb_ref, wab_ref, bias_ref, idx_ref, y_ref, z_ref):
    b = pl.program_id(0)
    r0 = pl.multiple_of(b * RK, RK)
    xr = x_ref[pl.ds(r0, RK), :]
    xrT = xT_ref[:, pl.ds(r0, RK)]
    sq_rT = jnp.sum(xrT * xrT, axis=0, keepdims=True)
    brT = bcolT_ref[:, pl.ds(r0, RK)]

    y_ref[...] = jnp.dot(xr, wb_ref[...], preferred_element_type=jnp.float32)
    z_ref[...] = (jnp.dot(xr, wab_ref[...], preferred_element_type=jnp.float32)
                  + bias_ref[...])

    topv0 = jnp.full((KP, RK), NEG, dtype=jnp.float32)
    topi0 = lax.broadcasted_iota(jnp.int32, (KP, RK), 0).astype(jnp.float32)

    def col_step(cb, carry):
        topv, topi = carry
        c0 = pl.multiple_of(cb * C, C)
        xc = x_ref[pl.ds(c0, C), :]
        sq_cT = jnp.sum(xc * xc, axis=1, keepdims=True)
        t0T = jnp.dot(xc, xrT, preferred_element_type=jnp.float32)
        distT = (sq_rT - 2.0 * t0T) + sq_cT
        bcT = batchF_ref[pl.ds(c0, C), :]
        sT = jnp.where(bcT == brT, -distT, NEG)
        gidT = (c0 + lax.broadcasted_iota(jnp.int32, (C, RK), 0)
                ).astype(jnp.float32)

        vals = jnp.concatenate([topv, sT], axis=0)
        gids = jnp.concatenate([topi, gidT], axis=0)
        return _extract_topk_rows(vals, gids)

    topv, topi = lax.fori_loop(cs_ref[b], ce_ref[b], col_step, (topv0, topi0))
    idx_ref[...] = topi[:K, :].astype(jnp.int32)


def _knn_and_linear(x_pad, xT, brow, bcolT, cs, ce, Wb, Wab, bias, fo):
    return pl.pallas_call(
        _knn_lin_body,
        grid=(NBK,),
        in_specs=[
            pl.BlockSpec(memory_space=pltpu.SMEM),
            pl.BlockSpec(memory_space=pltpu.SMEM),
            pl.BlockSpec((NP, DP), lambda b: (0, 0)),
            pl.BlockSpec((DP, NP), lambda b: (0, 0)),
            pl.BlockSpec((NP, 1), lambda b: (0, 0)),
            pl.BlockSpec((1, NP), lambda b: (0, 0)),
            pl.BlockSpec((DP, fo), lambda b: (0, 0)),
            pl.BlockSpec((DP, fo), lambda b: (0, 0)),
            pl.BlockSpec((1, fo), lambda b: (0, 0)),
        ],
        out_specs=[
            pl.BlockSpec((K, RK), lambda b: (0, b)),
            pl.BlockSpec((RK, fo), lambda b: (b, 0)),
            pl.BlockSpec((RK, fo), lambda b: (b, 0)),
        ],
        out_shape=[
            jax.ShapeDtypeStruct((K, NP), jnp.int32),
            jax.ShapeDtypeStruct((NP, fo), jnp.float32),
            jax.ShapeDtypeStruct((NP, fo), jnp.float32),
        ],
    )(cs, ce, x_pad, xT, brow, bcolT, Wb, Wab, bias)


def _gather_max_sc(y, z, idx_flat, fo):
    n_per_w = NP // SC_NW
    steps = n_per_w // SC_T
    ngrp = (SC_T * K) // SC_GRP
    fchunks = fo // SC_L
    mesh = plsc.VectorSubcoreMesh(core_axis_name="c", subcore_axis_name="s")

    @functools.partial(
        pl.kernel, mesh=mesh,
        out_type=jax.ShapeDtypeStruct((NP, fo), jnp.float32),
        scratch_types=[
            pltpu.VMEM((SC_T * K,), jnp.int32),
            pltpu.VMEM((SC_T * K, fo), jnp.float32),
            pltpu.VMEM((SC_T, fo), jnp.float32),
            pltpu.SemaphoreType.DMA,
        ],
    )
    def body(y_hbm, z_hbm, idx_hbm, out_hbm, idx_v, rows_v, acc_v, sem):
        wid = lax.axis_index("s") * SC_NC + lax.axis_index("c")
        base = wid * n_per_w

        def step(t, carry):
            n0 = base + t * SC_T
            pltpu.sync_copy(idx_hbm.at[pl.ds(n0 * K, SC_T * K)], idx_v)
            cps = [
                pltpu.async_copy(
                    y_hbm.at[idx_v.at[pl.ds(g * SC_GRP, SC_GRP)]],
                    rows_v.at[pl.ds(g * SC_GRP, SC_GRP)],
                    sem,
                )
                for g in range(ngrp)
            ]
            for cp in cps:
                cp.wait()
            pltpu.sync_copy(z_hbm.at[pl.ds(n0, SC_T)], acc_v)

            def inner(q, c2):
                n = q // fchunks
                f = (q % fchunks) * SC_L
                rbase = n * K
                a = rows_v[rbase, pl.ds(f, SC_L)]
                for kk in range(1, K):
                    a = jnp.maximum(a, rows_v[rbase + kk, pl.ds(f, SC_L)])
                acc_v[n, pl.ds(f, SC_L)] = acc_v[n, pl.ds(f, SC_L)] + a
                return c2

            lax.fori_loop(0, SC_T * fchunks, inner, 0)
            pltpu.sync_copy(acc_v, out_hbm.at[pl.ds(n0, SC_T)])
            return carry

        lax.fori_loop(0, steps, step, 0)

    return body(y, z, idx_flat)


def _linear_pool_body(slo_ref, shi_ref, cat_ref, lw_ref, lb_ref, brow_ref,
                      out_ref):
    b = pl.program_id(0)
    h = (jnp.dot(cat_ref[...], lw_ref[...], preferred_element_type=jnp.float32)
         + lb_ref[...])
    br = brow_ref[...]

    @pl.when(b == 0)
    def _():
        out_ref[...] = jnp.full(out_ref.shape, NEG, dtype=jnp.float32)

    def seg_step(s, carry):
        vals = jnp.where(br == s, h, NEG)
        m = jnp.max(vals, axis=0, keepdims=True)
        cur = out_ref[pl.ds(s, 1), :]
        out_ref[pl.ds(s, 1), :] = jnp.maximum(cur, m)
        return carry

    lax.fori_loop(slo_ref[b], shi_ref[b] + 1, seg_step, 0)


def _linear_pool(cat, lW, lb2, brow, slo, shi):
    return pl.pallas_call(
        _linear_pool_body,
        grid=(NB,),
        in_specs=[
            pl.BlockSpec(memory_space=pltpu.SMEM),
            pl.BlockSpec(memory_space=pltpu.SMEM),
            pl.BlockSpec((R, 512), lambda b: (b, 0)),
            pl.BlockSpec((512, 1024), lambda b: (0, 0)),
            pl.BlockSpec((1, 1024), lambda b: (0, 0)),
            pl.BlockSpec((R, 1), lambda b: (b, 0)),
        ],
        out_specs=pl.BlockSpec((NSEG, 1024), lambda b: (0, 0)),
        out_shape=jax.ShapeDtypeStruct((NSEG, 1024), jnp.float32),
        compiler_params=pltpu.CompilerParams(
            dimension_semantics=("arbitrary",)),
    )(slo, shi, cat, lW, lb2, brow)


def _head_body(p_ref, w1_ref, b1_ref, w2_ref, b2_ref, w3_ref, b3_ref, out_ref):
    p = p_ref[...]
    h = jnp.maximum(jnp.dot(p, w1_ref[...], preferred_element_type=jnp.float32)
                    + b1_ref[...], 0.0)
    h = jnp.maximum(jnp.dot(h, w2_ref[...], preferred_element_type=jnp.float32)
                    + b2_ref[...], 0.0)
    h = (jnp.dot(h, w3_ref[...], preferred_element_type=jnp.float32)
         + b3_ref[...])
    m = jnp.max(h, axis=1, keepdims=True)
    sh = h - m
    lse = jnp.log(jnp.sum(jnp.exp(sh), axis=1, keepdims=True))
    out_ref[...] = sh - lse


def _head(pooled, m1W, m1b, m2W, m2b, m3W, m3b):
    return pl.pallas_call(
        _head_body,
        out_shape=jax.ShapeDtypeStruct((NSEG, 40), jnp.float32),
    )(pooled, m1W, m1b.reshape(1, -1), m2W, m2b.reshape(1, -1),
      m3W, m3b.reshape(1, -1))


def _edge_layer(x_pad, brow, bcolT, cs, ce, W, bvec, fi, fo):
    fp = ((fo + 127) // 128) * 128
    Wa = W[:fi]
    Wb = W[fi:]
    Wb_p = jnp.zeros((DP, fp), jnp.float32).at[:fi, :fo].set(Wb)
    Wab_p = jnp.zeros((DP, fp), jnp.float32).at[:fi, :fo].set(Wa - Wb)
    bvec_p = jnp.zeros((1, fp), jnp.float32).at[0, :fo].set(bvec)
    xT = x_pad.T
    idx, y, z = _knn_and_linear(x_pad, xT, brow, bcolT, cs, ce,
                                Wb_p, Wab_p, bvec_p, fp)
    out = _gather_max_sc(y, z, idx.T.reshape(NP * K), fp)
    return out


def kernel(pos, batch, W1, b1, W2, b2, W3, b3, W4, b4, lW, lb,
           m1W, m1b, m2W, m2b, m3W, m3b):
    batch = batch.astype(jnp.int32)
    batch_pad = jnp.full((NP,), -1, jnp.int32).at[:N].set(batch)
    brow = batch_pad.reshape(NP, 1)
    bcolT = batch_pad.reshape(1, NP)

    seg_ids = jnp.arange(NSEG, dtype=jnp.int32)
    seg_start = jnp.searchsorted(batch, seg_ids, side="left").astype(jnp.int32)
    seg_end = jnp.searchsorted(batch, seg_ids, side="right").astype(jnp.int32)
    rbk = jnp.arange(NBK, dtype=jnp.int32) * RK
    firstk = batch[jnp.minimum(rbk, N - 1)]
    lastk = batch[jnp.minimum(rbk + RK - 1, N - 1)]
    cs = seg_start[firstk] // C
    ce = (seg_end[lastk] + C - 1) // C

    rb = jnp.arange(NB, dtype=jnp.int32) * R
    first = batch[jnp.minimum(rb, N - 1)]
    last = batch[jnp.minimum(rb + R - 1, N - 1)]

    x_pad = jnp.zeros((NP, DP), jnp.float32).at[:N, :3].set(pos)
    x1 = _edge_layer(x_pad, brow, bcolT, cs, ce, W1, b1, 3, 64)
    x2 = _edge_layer(x1, brow, bcolT, cs, ce, W2, b2, 64, 64)
    x3 = _edge_layer(x2, brow, bcolT, cs, ce, W3, b3, 64, 128)
    x4 = _edge_layer(x3, brow, bcolT, cs, ce, W4, b4, 128, 256)

    cat = jnp.concatenate([x1[:, :64], x2[:, :64], x3, x4], axis=1)
    pooled = _linear_pool(cat, lW, lb.reshape(1, -1), brow, first, last)
    return _head(pooled, m1W, m1b, m2W, m2b, m3W, m3b)


# speedup vs baseline: 2.7300x; 1.1159x over previous
import functools

import jax
import jax.numpy as jnp
from jax import lax
from jax.experimental import pallas as pl
from jax.experimental.pallas import tpu as pltpu
from jax.experimental.pallas import tpu_sc as plsc

N = 10000
K = 20
NSEG = 32
R = 256
RK = 256
C = 256
NP = 10240
NB = NP // R
NBK = NP // RK
DP = 128
NEG = float("-inf")
FBIG = 3.0e9

SC_NC = 2
SC_NS = 16
SC_L = 16
SC_NW = SC_NC * SC_NS
SC_T = 16
SC_GRP = 80


KP = 24


def _extract_topk_rows(vals, gids):
    nv, ni = [], []
    for _ in range(K):
        m = jnp.max(vals, axis=0, keepdims=True)
        eq = vals == m
        g = jnp.min(jnp.where(eq, gids, FBIG), axis=0, keepdims=True)
        nv.append(m)
        ni.append(g)
        vals = jnp.where(eq & (gids == g), NEG, vals)
    r = nv[0].shape[1]
    nv.append(jnp.full((KP - K, r), NEG, dtype=jnp.float32))
    ni.append(jnp.full((KP - K, r), FBIG, dtype=jnp.float32))
    return jnp.concatenate(nv, axis=0), jnp.concatenate(ni, axis=0)


def _knn_lin_body(cs_ref, ce_ref, x_ref, xT_ref, batchF_ref, bcolT_ref,
                  wb_ref, wab_ref, bias_ref, idx_ref, y_ref, z_ref):
    b = pl.program_id(0)
    r0 = pl.multiple_of(b * RK, RK)
    xr = x_ref[pl.ds(r0, RK), :]
    xrT = xT_ref[:, pl.ds(r0, RK)]
    sq_rT = jnp.sum(xrT * xrT, axis=0, keepdims=True)
    brT = bcolT_ref[:, pl.ds(r0, RK)]

    y_ref[...] = jnp.dot(xr, wb_ref[...], preferred_element_type=jnp.float32)
    z_ref[...] = (jnp.dot(xr, wab_ref[...], preferred_element_type=jnp.float32)
                  + bias_ref[...])

    topv0 = jnp.full((KP, RK), NEG, dtype=jnp.float32)
    topi0 = lax.broadcasted_iota(jnp.int32, (KP, RK), 0).astype(jnp.float32)

    def col_step(cb, carry):
        topv, topi = carry
        c0 = pl.multiple_of(cb * C, C)
        xc = x_ref[pl.ds(c0, C), :]
        sq_cT = jnp.sum(xc * xc, axis=1, keepdims=True)
        t0T = jnp.dot(xc, xrT, preferred_element_type=jnp.float32)
        distT = (sq_rT - 2.0 * t0T) + sq_cT
        bcT = batchF_ref[pl.ds(c0, C), :]
        sT = jnp.where(bcT == brT, -distT, NEG)
        gidT = (c0 + lax.broadcasted_iota(jnp.int32, (C, RK), 0)
                ).astype(jnp.float32)

        vals = jnp.concatenate([topv, sT], axis=0)
        gids = jnp.concatenate([topi, gidT], axis=0)
        return _extract_topk_rows(vals, gids)

    topv, topi = lax.fori_loop(cs_ref[b], ce_ref[b], col_step, (topv0, topi0))
    idx_ref[...] = topi[:K, :].astype(jnp.int32)


def _knn_and_linear(x_pad, xT, brow, bcolT, cs, ce, Wb, Wab, bias, fo):
    return pl.pallas_call(
        _knn_lin_body,
        grid=(NBK,),
        in_specs=[
            pl.BlockSpec(memory_space=pltpu.SMEM),
            pl.BlockSpec(memory_space=pltpu.SMEM),
            pl.BlockSpec((NP, DP), lambda b: (0, 0)),
            pl.BlockSpec((DP, NP), lambda b: (0, 0)),
            pl.BlockSpec((NP, 1), lambda b: (0, 0)),
            pl.BlockSpec((1, NP), lambda b: (0, 0)),
            pl.BlockSpec((DP, fo), lambda b: (0, 0)),
            pl.BlockSpec((DP, fo), lambda b: (0, 0)),
            pl.BlockSpec((1, fo), lambda b: (0, 0)),
        ],
        out_specs=[
            pl.BlockSpec((K, RK), lambda b: (0, b)),
            pl.BlockSpec((RK, fo), lambda b: (b, 0)),
            pl.BlockSpec((RK, fo), lambda b: (b, 0)),
        ],
        out_shape=[
            jax.ShapeDtypeStruct((K, NP), jnp.int32),
            jax.ShapeDtypeStruct((NP, fo), jnp.float32),
            jax.ShapeDtypeStruct((NP, fo), jnp.float32),
        ],
    )(cs, ce, x_pad, xT, brow, bcolT, Wb, Wab, bias)


def _gather_max_sc(y, z, idx_flat, fo):
    n_per_w = NP // SC_NW
    sc_t = SC_T if fo <= 128 else SC_T // 2
    steps = n_per_w // sc_t
    ngrp = (sc_t * K) // SC_GRP
    fchunks = fo // SC_L
    mesh = plsc.VectorSubcoreMesh(core_axis_name="c", subcore_axis_name="s")

    @functools.partial(
        pl.kernel, mesh=mesh,
        out_type=jax.ShapeDtypeStruct((NP, fo), jnp.float32),
        scratch_types=[
            pltpu.VMEM((sc_t * K,), jnp.int32),
            pltpu.VMEM((sc_t * K,), jnp.int32),
            pltpu.VMEM((sc_t * K, fo), jnp.float32),
            pltpu.VMEM((sc_t * K, fo), jnp.float32),
            pltpu.VMEM((sc_t, fo), jnp.float32),
            pltpu.SemaphoreType.DMA,
            pltpu.SemaphoreType.DMA,
        ],
    )
    def body(y_hbm, z_hbm, idx_hbm, out_hbm, idx0_v, idx1_v, rows0_v, rows1_v,
             acc_v, sem0, sem1):
        wid = lax.axis_index("s") * SC_NC + lax.axis_index("c")
        base = wid * n_per_w
        idxs = (idx0_v, idx1_v)
        rows = (rows0_v, rows1_v)
        sems = (sem0, sem1)

        def fire(t, buf):
            n0 = base + t * sc_t
            pltpu.sync_copy(idx_hbm.at[pl.ds(n0 * K, sc_t * K)], idxs[buf])
            for g in range(ngrp):
                pltpu.async_copy(
                    y_hbm.at[idxs[buf].at[pl.ds(g * SC_GRP, SC_GRP)]],
                    rows[buf].at[pl.ds(g * SC_GRP, SC_GRP)],
                    sems[buf],
                )

        def consume(t, buf):
            n0 = base + t * sc_t
            rows_v = rows[buf]
            for g in range(ngrp):
                pltpu.make_async_copy(
                    y_hbm.at[idxs[buf].at[pl.ds(g * SC_GRP, SC_GRP)]],
                    rows_v.at[pl.ds(g * SC_GRP, SC_GRP)],
                    sems[buf],
                ).wait()
            pltpu.sync_copy(z_hbm.at[pl.ds(n0, sc_t)], acc_v)

            def inner(q, c2):
                n = q // fchunks
                f = (q % fchunks) * SC_L
                rbase = n * K
                a = rows_v[rbase, pl.ds(f, SC_L)]
                for kk in range(1, K):
                    a = jnp.maximum(a, rows_v[rbase + kk, pl.ds(f, SC_L)])
                acc_v[n, pl.ds(f, SC_L)] = acc_v[n, pl.ds(f, SC_L)] + a
                return c2

            lax.fori_loop(0, sc_t * fchunks, inner, 0)
            pltpu.sync_copy(acc_v, out_hbm.at[pl.ds(n0, sc_t)])

        fire(0, 0)

        def pair(p, carry):
            t0 = p * 2
            fire(t0 + 1, 1)
            consume(t0, 0)

            @pl.when(t0 + 2 < steps)
            def _():
                fire(t0 + 2, 0)

            consume(t0 + 1, 1)
            return carry

        lax.fori_loop(0, steps // 2, pair, 0)

    return body(y, z, idx_flat)


def _linear_pool_body(slo_ref, shi_ref, cat_ref, lw_ref, lb_ref, brow_ref,
                      out_ref):
    b = pl.program_id(0)
    h = (jnp.dot(cat_ref[...], lw_ref[...], preferred_element_type=jnp.float32)
         + lb_ref[...])
    br = brow_ref[...]

    @pl.when(b == 0)
    def _():
        out_ref[...] = jnp.full(out_ref.shape, NEG, dtype=jnp.float32)

    def seg_step(s, carry):
        vals = jnp.where(br == s, h, NEG)
        m = jnp.max(vals, axis=0, keepdims=True)
        cur = out_ref[pl.ds(s, 1), :]
        out_ref[pl.ds(s, 1), :] = jnp.maximum(cur, m)
        return carry

    lax.fori_loop(slo_ref[b], shi_ref[b] + 1, seg_step, 0)


def _linear_pool(cat, lW, lb2, brow, slo, shi):
    return pl.pallas_call(
        _linear_pool_body,
        grid=(NB,),
        in_specs=[
            pl.BlockSpec(memory_space=pltpu.SMEM),
            pl.BlockSpec(memory_space=pltpu.SMEM),
            pl.BlockSpec((R, 512), lambda b: (b, 0)),
            pl.BlockSpec((512, 1024), lambda b: (0, 0)),
            pl.BlockSpec((1, 1024), lambda b: (0, 0)),
            pl.BlockSpec((R, 1), lambda b: (b, 0)),
        ],
        out_specs=pl.BlockSpec((NSEG, 1024), lambda b: (0, 0)),
        out_shape=jax.ShapeDtypeStruct((NSEG, 1024), jnp.float32),
        compiler_params=pltpu.CompilerParams(
            dimension_semantics=("arbitrary",)),
    )(slo, shi, cat, lW, lb2, brow)


def _head_body(p_ref, w1_ref, b1_ref, w2_ref, b2_ref, w3_ref, b3_ref, out_ref):
    p = p_ref[...]
    h = jnp.maximum(jnp.dot(p, w1_ref[...], preferred_element_type=jnp.float32)
                    + b1_ref[...], 0.0)
    h = jnp.maximum(jnp.dot(h, w2_ref[...], preferred_element_type=jnp.float32)
                    + b2_ref[...], 0.0)
    h = (jnp.dot(h, w3_ref[...], preferred_element_type=jnp.float32)
         + b3_ref[...])
    m = jnp.max(h, axis=1, keepdims=True)
    sh = h - m
    lse = jnp.log(jnp.sum(jnp.exp(sh), axis=1, keepdims=True))
    out_ref[...] = sh - lse


def _head(pooled, m1W, m1b, m2W, m2b, m3W, m3b):
    return pl.pallas_call(
        _head_body,
        out_shape=jax.ShapeDtypeStruct((NSEG, 40), jnp.float32),
    )(pooled, m1W, m1b.reshape(1, -1), m2W, m2b.reshape(1, -1),
      m3W, m3b.reshape(1, -1))


def _edge_layer(x_pad, brow, bcolT, cs, ce, W, bvec, fi, fo):
    fp = ((fo + 127) // 128) * 128
    Wa = W[:fi]
    Wb = W[fi:]
    Wb_p = jnp.zeros((DP, fp), jnp.float32).at[:fi, :fo].set(Wb)
    Wab_p = jnp.zeros((DP, fp), jnp.float32).at[:fi, :fo].set(Wa - Wb)
    bvec_p = jnp.zeros((1, fp), jnp.float32).at[0, :fo].set(bvec)
    xT = x_pad.T
    idx, y, z = _knn_and_linear(x_pad, xT, brow, bcolT, cs, ce,
                                Wb_p, Wab_p, bvec_p, fp)
    out = _gather_max_sc(y, z, idx.T.reshape(NP * K), fp)
    return out


def kernel(pos, batch, W1, b1, W2, b2, W3, b3, W4, b4, lW, lb,
           m1W, m1b, m2W, m2b, m3W, m3b):
    batch = batch.astype(jnp.int32)
    batch_pad = jnp.full((NP,), -1, jnp.int32).at[:N].set(batch)
    brow = batch_pad.reshape(NP, 1)
    bcolT = batch_pad.reshape(1, NP)

    seg_ids = jnp.arange(NSEG, dtype=jnp.int32)
    seg_start = jnp.searchsorted(batch, seg_ids, side="left").astype(jnp.int32)
    seg_end = jnp.searchsorted(batch, seg_ids, side="right").astype(jnp.int32)
    rbk = jnp.arange(NBK, dtype=jnp.int32) * RK
    firstk = batch[jnp.minimum(rbk, N - 1)]
    lastk = batch[jnp.minimum(rbk + RK - 1, N - 1)]
    cs = seg_start[firstk] // C
    ce = (seg_end[lastk] + C - 1) // C

    rb = jnp.arange(NB, dtype=jnp.int32) * R
    first = batch[jnp.minimum(rb, N - 1)]
    last = batch[jnp.minimum(rb + R - 1, N - 1)]

    x_pad = jnp.zeros((NP, DP), jnp.float32).at[:N, :3].set(pos)
    x1 = _edge_layer(x_pad, brow, bcolT, cs, ce, W1, b1, 3, 64)
    x2 = _edge_layer(x1, brow, bcolT, cs, ce, W2, b2, 64, 64)
    x3 = _edge_layer(x2, brow, bcolT, cs, ce, W3, b3, 64, 128)
    x4 = _edge_layer(x3, brow, bcolT, cs, ce, W4, b4, 128, 256)

    cat = jnp.concatenate([x1[:, :64], x2[:, :64], x3, x4], axis=1)
    pooled = _linear_pool(cat, lW, lb.reshape(1, -1), brow, first, last)
    return _head(pooled, m1W, m1b, m2W, m2b, m3W, m3b)


# speedup vs baseline: 2.8406x; 1.0405x over previous
import functools

import jax
import jax.numpy as jnp
from jax import lax
from jax.experimental import pallas as pl
from jax.experimental.pallas import tpu as pltpu
from jax.experimental.pallas import tpu_sc as plsc

N = 10000
K = 20
NSEG = 32
R = 256
RK = 256
C = 256
NP = 10240
NB = NP // R
NBK = NP // RK
DP = 128
NEG = float("-inf")
FBIG = 3.0e9

SC_NC = 2
SC_NS = 16
SC_L = 16
SC_NW = SC_NC * SC_NS
SC_T = 16
SC_GRP = 80


KP = 24


def _extract_topk_rows(vals, gids):
    nv, ni = [], []
    for _ in range(K):
        m = jnp.max(vals, axis=0, keepdims=True)
        eq = vals == m
        g = jnp.min(jnp.where(eq, gids, FBIG), axis=0, keepdims=True)
        nv.append(m)
        ni.append(g)
        vals = jnp.where(gids == g, NEG, vals)
    r = nv[0].shape[1]
    nv.append(jnp.full((KP - K, r), NEG, dtype=jnp.float32))
    ni.append(jnp.full((KP - K, r), FBIG, dtype=jnp.float32))
    return jnp.concatenate(nv, axis=0), jnp.concatenate(ni, axis=0)


def _knn_lin_body(cs_ref, ce_ref, x_ref, xT_ref, batchF_ref, bcolT_ref,
                  wb_ref, wab_ref, bias_ref, idx_ref, y_ref, z_ref):
    b = pl.program_id(0)
    r0 = pl.multiple_of(b * RK, RK)
    xr = x_ref[pl.ds(r0, RK), :]
    xrT = xT_ref[:, pl.ds(r0, RK)]
    sq_rT = jnp.sum(xrT * xrT, axis=0, keepdims=True)
    brT = bcolT_ref[:, pl.ds(r0, RK)]

    y_ref[...] = jnp.dot(xr, wb_ref[...], preferred_element_type=jnp.float32)
    z_ref[...] = (jnp.dot(xr, wab_ref[...], preferred_element_type=jnp.float32)
                  + bias_ref[...])

    topv0 = jnp.full((KP, RK), NEG, dtype=jnp.float32)
    topi0 = lax.broadcasted_iota(jnp.int32, (KP, RK), 0).astype(jnp.float32)

    def col_step(cb, carry):
        topv, topi = carry
        c0 = pl.multiple_of(cb * C, C)
        xc = x_ref[pl.ds(c0, C), :]
        sq_cT = jnp.sum(xc * xc, axis=1, keepdims=True)
        t0T = jnp.dot(xc, xrT, preferred_element_type=jnp.float32)
        distT = (sq_rT - 2.0 * t0T) + sq_cT
        bcT = batchF_ref[pl.ds(c0, C), :]
        sT = jnp.where(bcT == brT, -distT, NEG)
        gidT = (c0 + lax.broadcasted_iota(jnp.int32, (C, RK), 0)
                ).astype(jnp.float32)

        vals = jnp.concatenate([topv, sT], axis=0)
        gids = jnp.concatenate([topi, gidT], axis=0)
        return _extract_topk_rows(vals, gids)

    topv, topi = lax.fori_loop(cs_ref[b], ce_ref[b], col_step, (topv0, topi0))
    idx_ref[...] = topi[:K, :].astype(jnp.int32)


def _knn_and_linear(x_pad, xT, brow, bcolT, cs, ce, Wb, Wab, bias, fo):
    return pl.pallas_call(
        _knn_lin_body,
        grid=(NBK,),
        in_specs=[
            pl.BlockSpec(memory_space=pltpu.SMEM),
            pl.BlockSpec(memory_space=pltpu.SMEM),
            pl.BlockSpec((NP, DP), lambda b: (0, 0)),
            pl.BlockSpec((DP, NP), lambda b: (0, 0)),
            pl.BlockSpec((NP, 1), lambda b: (0, 0)),
            pl.BlockSpec((1, NP), lambda b: (0, 0)),
            pl.BlockSpec((DP, fo), lambda b: (0, 0)),
            pl.BlockSpec((DP, fo), lambda b: (0, 0)),
            pl.BlockSpec((1, fo), lambda b: (0, 0)),
        ],
        out_specs=[
            pl.BlockSpec((K, RK), lambda b: (0, b)),
            pl.BlockSpec((RK, fo), lambda b: (b, 0)),
            pl.BlockSpec((RK, fo), lambda b: (b, 0)),
        ],
        out_shape=[
            jax.ShapeDtypeStruct((K, NP), jnp.int32),
            jax.ShapeDtypeStruct((NP, fo), jnp.float32),
            jax.ShapeDtypeStruct((NP, fo), jnp.float32),
        ],
    )(cs, ce, x_pad, xT, brow, bcolT, Wb, Wab, bias)


def _gather_max_sc(y, z, idx_flat, fo):
    n_per_w = NP // SC_NW
    sc_t = SC_T if fo <= 128 else SC_T // 2
    steps = n_per_w // sc_t
    ngrp = (sc_t * K) // SC_GRP
    fchunks = fo // SC_L
    mesh = plsc.VectorSubcoreMesh(core_axis_name="c", subcore_axis_name="s")

    @functools.partial(
        pl.kernel, mesh=mesh,
        out_type=jax.ShapeDtypeStruct((NP, fo), jnp.float32),
        scratch_types=[
            pltpu.VMEM((sc_t * K,), jnp.int32),
            pltpu.VMEM((sc_t * K,), jnp.int32),
            pltpu.VMEM((sc_t * K, fo), jnp.float32),
            pltpu.VMEM((sc_t * K, fo), jnp.float32),
            pltpu.VMEM((sc_t, fo), jnp.float32),
            pltpu.SemaphoreType.DMA,
            pltpu.SemaphoreType.DMA,
        ],
    )
    def body(y_hbm, z_hbm, idx_hbm, out_hbm, idx0_v, idx1_v, rows0_v, rows1_v,
             acc_v, sem0, sem1):
        wid = lax.axis_index("s") * SC_NC + lax.axis_index("c")
        base = wid * n_per_w
        idxs = (idx0_v, idx1_v)
        rows = (rows0_v, rows1_v)
        sems = (sem0, sem1)

        def fire(t, buf):
            n0 = base + t * sc_t
            pltpu.sync_copy(idx_hbm.at[pl.ds(n0 * K, sc_t * K)], idxs[buf])
            for g in range(ngrp):
                pltpu.async_copy(
                    y_hbm.at[idxs[buf].at[pl.ds(g * SC_GRP, SC_GRP)]],
                    rows[buf].at[pl.ds(g * SC_GRP, SC_GRP)],
                    sems[buf],
                )

        def consume(t, buf):
            n0 = base + t * sc_t
            rows_v = rows[buf]
            for g in range(ngrp):
                pltpu.make_async_copy(
                    y_hbm.at[idxs[buf].at[pl.ds(g * SC_GRP, SC_GRP)]],
                    rows_v.at[pl.ds(g * SC_GRP, SC_GRP)],
                    sems[buf],
                ).wait()
            pltpu.sync_copy(z_hbm.at[pl.ds(n0, sc_t)], acc_v)

            def inner(q, c2):
                n = q // fchunks
                f = (q % fchunks) * SC_L
                rbase = n * K
                a = rows_v[rbase, pl.ds(f, SC_L)]
                for kk in range(1, K):
                    a = jnp.maximum(a, rows_v[rbase + kk, pl.ds(f, SC_L)])
                acc_v[n, pl.ds(f, SC_L)] = acc_v[n, pl.ds(f, SC_L)] + a
                return c2

            lax.fori_loop(0, sc_t * fchunks, inner, 0)
            pltpu.sync_copy(acc_v, out_hbm.at[pl.ds(n0, sc_t)])

        fire(0, 0)

        def pair(p, carry):
            t0 = p * 2
            fire(t0 + 1, 1)
            consume(t0, 0)

            @pl.when(t0 + 2 < steps)
            def _():
                fire(t0 + 2, 0)

            consume(t0 + 1, 1)
            return carry

        lax.fori_loop(0, steps // 2, pair, 0)

    return body(y, z, idx_flat)


def _linear_pool_body(slo_ref, shi_ref, cat_ref, lw_ref, lb_ref, brow_ref,
                      out_ref):
    b = pl.program_id(0)
    h = (jnp.dot(cat_ref[...], lw_ref[...], preferred_element_type=jnp.float32)
         + lb_ref[...])
    br = brow_ref[...]

    @pl.when(b == 0)
    def _():
        out_ref[...] = jnp.full(out_ref.shape, NEG, dtype=jnp.float32)

    def seg_step(s, carry):
        vals = jnp.where(br == s, h, NEG)
        m = jnp.max(vals, axis=0, keepdims=True)
        cur = out_ref[pl.ds(s, 1), :]
        out_ref[pl.ds(s, 1), :] = jnp.maximum(cur, m)
        return carry

    lax.fori_loop(slo_ref[b], shi_ref[b] + 1, seg_step, 0)


def _linear_pool(cat, lW, lb2, brow, slo, shi):
    return pl.pallas_call(
        _linear_pool_body,
        grid=(NB,),
        in_specs=[
            pl.BlockSpec(memory_space=pltpu.SMEM),
            pl.BlockSpec(memory_space=pltpu.SMEM),
            pl.BlockSpec((R, 512), lambda b: (b, 0)),
            pl.BlockSpec((512, 1024), lambda b: (0, 0)),
            pl.BlockSpec((1, 1024), lambda b: (0, 0)),
            pl.BlockSpec((R, 1), lambda b: (b, 0)),
        ],
        out_specs=pl.BlockSpec((NSEG, 1024), lambda b: (0, 0)),
        out_shape=jax.ShapeDtypeStruct((NSEG, 1024), jnp.float32),
        compiler_params=pltpu.CompilerParams(
            dimension_semantics=("arbitrary",)),
    )(slo, shi, cat, lW, lb2, brow)


def _head_body(p_ref, w1_ref, b1_ref, w2_ref, b2_ref, w3_ref, b3_ref, out_ref):
    p = p_ref[...]
    h = jnp.maximum(jnp.dot(p, w1_ref[...], preferred_element_type=jnp.float32)
                    + b1_ref[...], 0.0)
    h = jnp.maximum(jnp.dot(h, w2_ref[...], preferred_element_type=jnp.float32)
                    + b2_ref[...], 0.0)
    h = (jnp.dot(h, w3_ref[...], preferred_element_type=jnp.float32)
         + b3_ref[...])
    m = jnp.max(h, axis=1, keepdims=True)
    sh = h - m
    lse = jnp.log(jnp.sum(jnp.exp(sh), axis=1, keepdims=True))
    out_ref[...] = sh - lse


def _head(pooled, m1W, m1b, m2W, m2b, m3W, m3b):
    return pl.pallas_call(
        _head_body,
        out_shape=jax.ShapeDtypeStruct((NSEG, 40), jnp.float32),
    )(pooled, m1W, m1b.reshape(1, -1), m2W, m2b.reshape(1, -1),
      m3W, m3b.reshape(1, -1))


def _edge_layer(x_pad, brow, bcolT, cs, ce, W, bvec, fi, fo):
    fp = ((fo + 127) // 128) * 128
    Wa = W[:fi]
    Wb = W[fi:]
    Wb_p = jnp.zeros((DP, fp), jnp.float32).at[:fi, :fo].set(Wb)
    Wab_p = jnp.zeros((DP, fp), jnp.float32).at[:fi, :fo].set(Wa - Wb)
    bvec_p = jnp.zeros((1, fp), jnp.float32).at[0, :fo].set(bvec)
    xT = x_pad.T
    idx, y, z = _knn_and_linear(x_pad, xT, brow, bcolT, cs, ce,
                                Wb_p, Wab_p, bvec_p, fp)
    out = _gather_max_sc(y, z, idx.T.reshape(NP * K), fp)
    return out


def kernel(pos, batch, W1, b1, W2, b2, W3, b3, W4, b4, lW, lb,
           m1W, m1b, m2W, m2b, m3W, m3b):
    batch = batch.astype(jnp.int32)
    batch_pad = jnp.full((NP,), -1, jnp.int32).at[:N].set(batch)
    brow = batch_pad.reshape(NP, 1)
    bcolT = batch_pad.reshape(1, NP)

    seg_ids = jnp.arange(NSEG, dtype=jnp.int32)
    seg_start = jnp.searchsorted(batch, seg_ids, side="left").astype(jnp.int32)
    seg_end = jnp.searchsorted(batch, seg_ids, side="right").astype(jnp.int32)
    rbk = jnp.arange(NBK, dtype=jnp.int32) * RK
    firstk = batch[jnp.minimum(rbk, N - 1)]
    lastk = batch[jnp.minimum(rbk + RK - 1, N - 1)]
    cs = seg_start[firstk] // C
    ce = (seg_end[lastk] + C - 1) // C

    rb = jnp.arange(NB, dtype=jnp.int32) * R
    first = batch[jnp.minimum(rb, N - 1)]
    last = batch[jnp.minimum(rb + R - 1, N - 1)]

    x_pad = jnp.zeros((NP, DP), jnp.float32).at[:N, :3].set(pos)
    x1 = _edge_layer(x_pad, brow, bcolT, cs, ce, W1, b1, 3, 64)
    x2 = _edge_layer(x1, brow, bcolT, cs, ce, W2, b2, 64, 64)
    x3 = _edge_layer(x2, brow, bcolT, cs, ce, W3, b3, 64, 128)
    x4 = _edge_layer(x3, brow, bcolT, cs, ce, W4, b4, 128, 256)

    cat = jnp.concatenate([x1[:, :64], x2[:, :64], x3, x4], axis=1)
    pooled = _linear_pool(cat, lW, lb.reshape(1, -1), brow, first, last)
    return _head(pooled, m1W, m1b, m2W, m2b, m3W, m3b)
